# baseline (speedup 1.0000x reference)
import functools

import jax
import jax.numpy as jnp
from jax import lax
from jax.experimental import pallas as pl
from jax.experimental.pallas import tpu as pltpu
from jax.experimental.pallas import tpu_sc as plsc

_B, _S, _D = 4, 8192, 1024
_NC, _NS = 2, 16
_NW = _NC * _NS
_RPW = (_B * _S) // _NW
_CR = 32
_CE = _CR * _D
_NCHUNK = _RPW // _CR


def _sc_add_body(in_hbm, pos_hbm, out_hbm, in_v, pos_v, sem_a, sem_b):
    wid = lax.axis_index("s") * _NC + lax.axis_index("c")
    base = wid * (_RPW * _D)
    pos_base = (wid % (_S // _RPW)) * (_RPW * _D)

    def chunk(i, _):
        off = base + i * _CE
        poff = pos_base + i * _CE
        ca = pltpu.async_copy(in_hbm.at[pl.ds(off, _CE)], in_v, sem_a)
        cb = pltpu.async_copy(pos_hbm.at[pl.ds(poff, _CE)], pos_v, sem_b)
        ca.wait()
        cb.wait()

        def add16(j, _):
            k = j * 64
            for u in range(4):
                sl = pl.ds(k + u * 16, 16)
                in_v[sl] = in_v[sl] + pos_v[sl]
            return 0

        lax.fori_loop(0, _CE // 64, add16, 0, unroll=4)
        pltpu.sync_copy(in_v, out_hbm.at[pl.ds(off, _CE)])
        return 0

    lax.fori_loop(0, _NCHUNK, chunk, 0)


@functools.partial(
    pl.kernel,
    mesh=plsc.VectorSubcoreMesh(core_axis_name="c", subcore_axis_name="s"),
    out_type=jax.ShapeDtypeStruct((_B * _S * _D,), jnp.float32),
    scratch_types=[
        pltpu.VMEM((_CE,), jnp.float32),
        pltpu.VMEM((_CE,), jnp.float32),
        pltpu.SemaphoreType.DMA,
        pltpu.SemaphoreType.DMA,
    ],
)
def _sc_add(in_hbm, pos_hbm, out_hbm, in_v, pos_v, sem_a, sem_b):
    _sc_add_body(in_hbm, pos_hbm, out_hbm, in_v, pos_v, sem_a, sem_b)


def kernel(inputs, pos_table):
    B, S, D = inputs.shape
    flat_in = inputs.reshape(B * S * D)
    flat_pos = pos_table[:S].reshape(S * D)
    out = _sc_add(flat_in, flat_pos)
    return out.reshape(B, S, D)


# speedup vs baseline: 1.7209x; 1.7209x over previous
import functools

import jax
import jax.numpy as jnp
from jax import lax
from jax.experimental import pallas as pl
from jax.experimental.pallas import tpu as pltpu
from jax.experimental.pallas import tpu_sc as plsc

_B, _S, _D = 4, 8192, 1024
_NC, _NS = 2, 16
_NW = _NC * _NS
_RPW = (_B * _S) // _NW
_CR = 16
_CE = _CR * _D
_NCHUNK = _RPW // _CR


def _sc_add_body(in_hbm, pos_hbm, out_hbm, in_v, pos_v, sem_in, sem_pos,
                 sem_out):
    wid = lax.axis_index("s") * _NC + lax.axis_index("c")
    base = wid * (_RPW * _D)
    pos_base = (wid % (_S // _RPW)) * (_RPW * _D)

    def start_load(g, p):
        pltpu.async_copy(in_hbm.at[pl.ds(base + g * _CE, _CE)], in_v[p],
                         sem_in[p])
        pltpu.async_copy(pos_hbm.at[pl.ds(pos_base + g * _CE, _CE)], pos_v[p],
                         sem_pos[p])

    def wait_load(g, p):
        pltpu.make_async_copy(in_hbm.at[pl.ds(base + g * _CE, _CE)], in_v[p],
                              sem_in[p]).wait()
        pltpu.make_async_copy(pos_hbm.at[pl.ds(pos_base + g * _CE, _CE)],
                              pos_v[p], sem_pos[p]).wait()

    start_load(0, 0)
    start_load(1, 1)

    def pair(t, _):
        for p in range(2):
            g = 2 * t + p
            wait_load(g, p)

            @plsc.parallel_loop(0, _CE, step=16, unroll=8)
            def _add(k):
                sl = pl.ds(k, 16)
                in_v[p][sl] = in_v[p][sl] + pos_v[p][sl]

            copy_out = pltpu.async_copy(in_v[p],
                                        out_hbm.at[pl.ds(base + g * _CE, _CE)],
                                        sem_out[p])
            copy_out.wait()

            @pl.when(g + 2 < _NCHUNK)
            def _():
                start_load(g + 2, p)
        return 0

    lax.fori_loop(0, _NCHUNK // 2, pair, 0)


@functools.partial(
    pl.kernel,
    mesh=plsc.VectorSubcoreMesh(core_axis_name="c", subcore_axis_name="s"),
    out_type=jax.ShapeDtypeStruct((_B * _S * _D,), jnp.float32),
    scratch_types=[
        pltpu.VMEM((_CE,), jnp.float32),
        pltpu.VMEM((_CE,), jnp.float32),
        pltpu.VMEM((_CE,), jnp.float32),
        pltpu.VMEM((_CE,), jnp.float32),
        pltpu.SemaphoreType.DMA,
        pltpu.SemaphoreType.DMA,
        pltpu.SemaphoreType.DMA,
        pltpu.SemaphoreType.DMA,
        pltpu.SemaphoreType.DMA,
        pltpu.SemaphoreType.DMA,
    ],
)
def _sc_add(in_hbm, pos_hbm, out_hbm, in_v0, in_v1, pos_v0, pos_v1,
            si0, si1, sp0, sp1, so0, so1):
    _sc_add_body(in_hbm, pos_hbm, out_hbm, [in_v0, in_v1], [pos_v0, pos_v1],
                 [si0, si1], [sp0, sp1], [so0, so1])


def kernel(inputs, pos_table):
    B, S, D = inputs.shape
    flat_in = inputs.reshape(B * S * D)
    flat_pos = pos_table[:S].reshape(S * D)
    out = _sc_add(flat_in, flat_pos)
    return out.reshape(B, S, D)


# speedup vs baseline: 4.4531x; 2.5877x over previous
import functools

import jax
import jax.numpy as jnp
from jax import lax
from jax.experimental import pallas as pl
from jax.experimental.pallas import tpu as pltpu
from jax.experimental.pallas import tpu_sc as plsc

_B, _S, _D = 4, 8192, 1024
_NC, _NS = 2, 16
_NW = _NC * _NS
_RPW = (_B * _S) // _NW
_CR = 16
_NCHUNK = _RPW // _CR


def _sc_add_body(in_hbm, pos_hbm, out_hbm, in_v, pos_v, sem_in, sem_pos,
                 sem_out):
    wid = lax.axis_index("s") * _NC + lax.axis_index("c")
    base = wid * _RPW
    pos_base = (wid % (_S // _RPW)) * _RPW

    def start_load(g, p):
        pltpu.async_copy(in_hbm.at[pl.ds(base + g * _CR, _CR), :], in_v[p],
                         sem_in[p])
        pltpu.async_copy(pos_hbm.at[pl.ds(pos_base + g * _CR, _CR), :],
                         pos_v[p], sem_pos[p])

    def wait_load(g, p):
        pltpu.make_async_copy(in_hbm.at[pl.ds(base + g * _CR, _CR), :],
                              in_v[p], sem_in[p]).wait()
        pltpu.make_async_copy(pos_hbm.at[pl.ds(pos_base + g * _CR, _CR), :],
                              pos_v[p], sem_pos[p]).wait()

    start_load(0, 0)
    start_load(1, 1)

    def pair(t, _):
        for p in range(2):
            g = 2 * t + p
            wait_load(g, p)

            @plsc.parallel_loop(0, _CR * _D // 16, step=1, unroll=8)
            def _add(k):
                r = k >> 6
                c = (k & 63) * 16
                sl = pl.ds(c, 16)
                in_v[p][r, sl] = in_v[p][r, sl] + pos_v[p][r, sl]

            copy_out = pltpu.async_copy(
                in_v[p], out_hbm.at[pl.ds(base + g * _CR, _CR), :], sem_out[p])
            copy_out.wait()

            @pl.when(g + 2 < _NCHUNK)
            def _():
                start_load(g + 2, p)
        return 0

    lax.fori_loop(0, _NCHUNK // 2, pair, 0)


@functools.partial(
    pl.kernel,
    mesh=plsc.VectorSubcoreMesh(core_axis_name="c", subcore_axis_name="s"),
    out_type=jax.ShapeDtypeStruct((_B * _S, _D), jnp.float32),
    scratch_types=[
        pltpu.VMEM((_CR, _D), jnp.float32),
        pltpu.VMEM((_CR, _D), jnp.float32),
        pltpu.VMEM((_CR, _D), jnp.float32),
        pltpu.VMEM((_CR, _D), jnp.float32),
        pltpu.SemaphoreType.DMA,
        pltpu.SemaphoreType.DMA,
        pltpu.SemaphoreType.DMA,
        pltpu.SemaphoreType.DMA,
        pltpu.SemaphoreType.DMA,
        pltpu.SemaphoreType.DMA,
    ],
)
def _sc_add(in_hbm, pos_hbm, out_hbm, in_v0, in_v1, pos_v0, pos_v1,
            si0, si1, sp0, sp1, so0, so1):
    _sc_add_body(in_hbm, pos_hbm, out_hbm, [in_v0, in_v1], [pos_v0, pos_v1],
                 [si0, si1], [sp0, sp1], [so0, so1])


def kernel(inputs, pos_table):
    B, S, D = inputs.shape
    out = _sc_add(inputs.reshape(B * S, D), pos_table[:S])
    return out.reshape(B, S, D)


# speedup vs baseline: 5.8541x; 1.3146x over previous
import functools

import jax
import jax.numpy as jnp
from jax import lax
from jax.experimental import pallas as pl
from jax.experimental.pallas import tpu as pltpu
from jax.experimental.pallas import tpu_sc as plsc

_B, _S, _D = 4, 8192, 1024
_NC, _NS = 2, 16
_NW = _NC * _NS
_RPW = _S // _NW
_CR = 8
_NCHUNK = _RPW // _CR


def _sc_add_body(in_hbm, pos_hbm, out_hbm, in_v, pos_v, sem_in, sem_pos,
                 sem_out):
    wid = lax.axis_index("s") * _NC + lax.axis_index("c")
    pos_base = wid * _RPW

    def start_load(g, p):
        r0 = pos_base + g * _CR
        pltpu.async_copy(pos_hbm.at[pl.ds(r0, _CR), :], pos_v[p], sem_pos[p])
        for b in range(_B):
            pltpu.async_copy(in_hbm.at[pl.ds(b * _S + r0, _CR), :],
                             in_v[b][p], sem_in[p])

    def wait_load(g, p):
        r0 = pos_base + g * _CR
        pltpu.make_async_copy(pos_hbm.at[pl.ds(r0, _CR), :], pos_v[p],
                              sem_pos[p]).wait()
        for b in range(_B):
            pltpu.make_async_copy(in_hbm.at[pl.ds(b * _S + r0, _CR), :],
                                  in_v[b][p], sem_in[p]).wait()

    start_load(0, 0)
    start_load(1, 1)

    def pair(t, _):
        for p in range(2):
            g = 2 * t + p
            r0 = pos_base + g * _CR
            wait_load(g, p)

            @plsc.parallel_loop(0, _CR * _D // 16, step=1, unroll=4)
            def _add(k):
                r = k >> 6
                sl = pl.ds((k & 63) * 16, 16)
                pv = pos_v[p][r, sl]
                for b in range(_B):
                    in_v[b][p][r, sl] = in_v[b][p][r, sl] + pv

            copies = [
                pltpu.async_copy(in_v[b][p],
                                 out_hbm.at[pl.ds(b * _S + r0, _CR), :],
                                 sem_out[p])
                for b in range(_B)
            ]
            for c in copies:
                c.wait()

            @pl.when(g + 2 < _NCHUNK)
            def _():
                start_load(g + 2, p)
        return 0

    lax.fori_loop(0, _NCHUNK // 2, pair, 0)


@functools.partial(
    pl.kernel,
    mesh=plsc.VectorSubcoreMesh(core_axis_name="c", subcore_axis_name="s"),
    out_type=jax.ShapeDtypeStruct((_B * _S, _D), jnp.float32),
    scratch_types=(
        [pltpu.VMEM((_CR, _D), jnp.float32)] * (2 * _B + 2)
        + [pltpu.SemaphoreType.DMA] * 6
    ),
)
def _sc_add(in_hbm, pos_hbm, out_hbm,
            i00, i01, i10, i11, i20, i21, i30, i31, p0, p1,
            si0, si1, sp0, sp1, so0, so1):
    _sc_add_body(
        in_hbm, pos_hbm, out_hbm,
        [[i00, i01], [i10, i11], [i20, i21], [i30, i31]], [p0, p1],
        [si0, si1], [sp0, sp1], [so0, so1])


def kernel(inputs, pos_table):
    B, S, D = inputs.shape
    out = _sc_add(inputs.reshape(B * S, D), pos_table[:S])
    return out.reshape(B, S, D)


# speedup vs baseline: 5.8680x; 1.0024x over previous
import functools

import jax
import jax.numpy as jnp
from jax import lax
from jax.experimental import pallas as pl
from jax.experimental.pallas import tpu as pltpu
from jax.experimental.pallas import tpu_sc as plsc

_B, _S, _D = 4, 8192, 1024
_NC, _NS = 2, 16
_NW = _NC * _NS
_RPW = _S // _NW
_CR = 8
_NCHUNK = _RPW // _CR
_RING = 3


def _sc_add_body(in_hbm, pos_hbm, out_hbm, in_v, pos_v, sem_in, sem_pos,
                 sem_out):
    wid = lax.axis_index("s") * _NC + lax.axis_index("c")
    pos_base = wid * _RPW

    def start_load(g, sl):
        r0 = pos_base + g * _CR
        pltpu.async_copy(pos_hbm.at[pl.ds(r0, _CR), :], pos_v[sl],
                         sem_pos[sl])
        for b in range(_B):
            pltpu.async_copy(in_hbm.at[pl.ds(b * _S + r0, _CR), :],
                             in_v[b][sl], sem_in[sl])

    def wait_load(g, sl):
        r0 = pos_base + g * _CR
        pltpu.make_async_copy(pos_hbm.at[pl.ds(r0, _CR), :], pos_v[sl],
                              sem_pos[sl]).wait()
        for b in range(_B):
            pltpu.make_async_copy(in_hbm.at[pl.ds(b * _S + r0, _CR), :],
                                  in_v[b][sl], sem_in[sl]).wait()

    def start_store(g, sl):
        r0 = pos_base + g * _CR
        for b in range(_B):
            pltpu.async_copy(in_v[b][sl],
                             out_hbm.at[pl.ds(b * _S + r0, _CR), :],
                             sem_out[sl])

    def wait_store(g, sl):
        r0 = pos_base + g * _CR
        for b in range(_B):
            pltpu.make_async_copy(in_v[b][sl],
                                  out_hbm.at[pl.ds(b * _S + r0, _CR), :],
                                  sem_out[sl]).wait()

    def compute(sl):
        @plsc.parallel_loop(0, _CR * _D // 16, step=1, unroll=4)
        def _add(k):
            r = k >> 6
            cs = pl.ds((k & 63) * 16, 16)
            pv = pos_v[sl][r, cs]
            for b in range(_B):
                in_v[b][sl][r, cs] = in_v[b][sl][r, cs] + pv

    def step(g, sl, tail=False):
        wait_load(g, sl)
        compute(sl)
        start_store(g, sl)
        if tail:
            wait_store(g - 1, (sl - 1) % _RING)
        else:
            @pl.when(g >= 1)
            def _():
                wait_store(g - 1, (sl - 1) % _RING)

            @pl.when(g + 2 < _NCHUNK)
            def _():
                start_load(g + 2, (sl + 2) % _RING)

    start_load(0, 0)
    start_load(1, 1)
    n_main = (_NCHUNK // _RING) * _RING

    def triple(t, _):
        for p in range(_RING):
            step(t * _RING + p, p)
        return 0

    lax.fori_loop(0, n_main // _RING, triple, 0)
    for g in range(n_main, _NCHUNK):
        step(g, g % _RING, tail=True)
    wait_store(_NCHUNK - 1, (_NCHUNK - 1) % _RING)


@functools.partial(
    pl.kernel,
    mesh=plsc.VectorSubcoreMesh(core_axis_name="c", subcore_axis_name="s"),
    out_type=jax.ShapeDtypeStruct((_B * _S, _D), jnp.float32),
    scratch_types=(
        [pltpu.VMEM((_CR, _D), jnp.float32)] * (_RING * (_B + 1))
        + [pltpu.SemaphoreType.DMA] * (3 * _RING)
    ),
)
def _sc_add(in_hbm, pos_hbm, out_hbm,
            i00, i01, i02, i10, i11, i12, i20, i21, i22, i30, i31, i32,
            p0, p1, p2,
            si0, si1, si2, sp0, sp1, sp2, so0, so1, so2):
    _sc_add_body(
        in_hbm, pos_hbm, out_hbm,
        [[i00, i01, i02], [i10, i11, i12], [i20, i21, i22], [i30, i31, i32]],
        [p0, p1, p2],
        [si0, si1, si2], [sp0, sp1, sp2], [so0, so1, so2])


def kernel(inputs, pos_table):
    B, S, D = inputs.shape
    out = _sc_add(inputs.reshape(B * S, D), pos_table[:S])
    return out.reshape(B, S, D)


# speedup vs baseline: 5.8690x; 1.0002x over previous
import functools

import jax
import jax.numpy as jnp
from jax import lax
from jax.experimental import pallas as pl
from jax.experimental.pallas import tpu as pltpu
from jax.experimental.pallas import tpu_sc as plsc

_B, _S, _D = 4, 8192, 1024
_NC, _NS = 2, 16
_NW = _NC * _NS
_RPW = _S // _NW
_CR = 8
_NCHUNK = _RPW // _CR
_RING = 3


def _sc_add_body(in_hbm, pos_hbm, out_hbm, in_v, pos_v, sem_in, sem_pos,
                 sem_out):
    wid = lax.axis_index("s") * _NC + lax.axis_index("c")
    pos_base = wid * _RPW

    def start_load(g, sl):
        r0 = pos_base + g * _CR
        pltpu.async_copy(pos_hbm.at[pl.ds(r0, _CR), :], pos_v[sl],
                         sem_pos[sl])
        for b in range(_B):
            pltpu.async_copy(in_hbm.at[pl.ds(b * _S + r0, _CR), :],
                             in_v[b][sl], sem_in[sl])

    def wait_load(g, sl):
        r0 = pos_base + g * _CR
        pltpu.make_async_copy(pos_hbm.at[pl.ds(r0, _CR), :], pos_v[sl],
                              sem_pos[sl]).wait()
        for b in range(_B):
            pltpu.make_async_copy(in_hbm.at[pl.ds(b * _S + r0, _CR), :],
                                  in_v[b][sl], sem_in[sl]).wait()

    def start_store(g, sl):
        r0 = pos_base + g * _CR
        for b in range(_B):
            pltpu.async_copy(in_v[b][sl],
                             out_hbm.at[pl.ds(b * _S + r0, _CR), :],
                             sem_out[sl])

    def wait_store(g, sl):
        r0 = pos_base + g * _CR
        for b in range(_B):
            pltpu.make_async_copy(in_v[b][sl],
                                  out_hbm.at[pl.ds(b * _S + r0, _CR), :],
                                  sem_out[sl]).wait()

    def compute(sl):
        @plsc.parallel_loop(0, _CR * _D // 16, step=1, unroll=8)
        def _add(k):
            r = k >> 6
            cs = pl.ds((k & 63) * 16, 16)
            pv = pos_v[sl][r, cs]
            for b in range(_B):
                in_v[b][sl][r, cs] = in_v[b][sl][r, cs] + pv

    def step(g, sl, tail=False):
        wait_load(g, sl)
        compute(sl)
        start_store(g, sl)
        if tail:
            wait_store(g - 1, (sl - 1) % _RING)
        else:
            @pl.when(g >= 1)
            def _():
                wait_store(g - 1, (sl - 1) % _RING)

            @pl.when(g + 2 < _NCHUNK)
            def _():
                start_load(g + 2, (sl + 2) % _RING)

    start_load(0, 0)
    start_load(1, 1)
    n_main = (_NCHUNK // _RING) * _RING

    def triple(t, _):
        for p in range(_RING):
            step(t * _RING + p, p)
        return 0

    lax.fori_loop(0, n_main // _RING, triple, 0)
    for g in range(n_main, _NCHUNK):
        step(g, g % _RING, tail=True)
    wait_store(_NCHUNK - 1, (_NCHUNK - 1) % _RING)


@functools.partial(
    pl.kernel,
    mesh=plsc.VectorSubcoreMesh(core_axis_name="c", subcore_axis_name="s"),
    out_type=jax.ShapeDtypeStruct((_B * _S, _D), jnp.float32),
    scratch_types=(
        [pltpu.VMEM((_CR, _D), jnp.float32)] * (_RING * (_B + 1))
        + [pltpu.SemaphoreType.DMA] * (3 * _RING)
    ),
)
def _sc_add(in_hbm, pos_hbm, out_hbm,
            i00, i01, i02, i10, i11, i12, i20, i21, i22, i30, i31, i32,
            p0, p1, p2,
            si0, si1, si2, sp0, sp1, sp2, so0, so1, so2):
    _sc_add_body(
        in_hbm, pos_hbm, out_hbm,
        [[i00, i01, i02], [i10, i11, i12], [i20, i21, i22], [i30, i31, i32]],
        [p0, p1, p2],
        [si0, si1, si2], [sp0, sp1, sp2], [so0, so1, so2])


def kernel(inputs, pos_table):
    B, S, D = inputs.shape
    out = _sc_add(inputs.reshape(B * S, D), pos_table[:S])
    return out.reshape(B, S, D)
